# baseline (speedup 1.0000x reference)
import jax
import jax.numpy as jnp
from jax.experimental import pallas as pl
from jax.experimental.pallas import tpu as pltpu

DIM = 2048
HEADS = 16
HD = 128
MLPH = 8192
TL = 256
IL = 1024
S = TL + IL
TAU = 5.0
BF = jnp.bfloat16
F32 = jnp.float32

_INTERPRET = False


def _pc(*args, **kwargs):
    return pl.pallas_call(*args, interpret=_INTERPRET, **kwargs)


def _bdot(a, b):
    return jnp.dot(a.astype(BF), b.astype(BF), preferred_element_type=F32)


def _adaln_kernel(temb_ref, w_ref, b_ref, out_ref):
    t = jax.nn.silu(temb_ref[...])
    t8 = jnp.concatenate([t, t, t, t], axis=0)
    r = _bdot(t8, w_ref[...])
    out_ref[...] = r[:2, :] + b_ref[...]


def _adaln(temb, adaln_w, adaln_b):
    NB = 512
    n = (3 * DIM) // NB
    return _pc(
        _adaln_kernel,
        grid=(n,),
        in_specs=[
            pl.BlockSpec((2, DIM), lambda i: (0, 0)),
            pl.BlockSpec((DIM, NB), lambda i: (0, i)),
            pl.BlockSpec((1, NB), lambda i: (0, i)),
        ],
        out_specs=pl.BlockSpec((2, NB), lambda i: (0, i)),
        out_shape=jax.ShapeDtypeStruct((2, 3 * DIM), F32),
    )(temb, adaln_w, adaln_b.reshape(1, 3 * DIM))


def _ln_kernel(hid_ref, emb_ref, out_ref):
    x = hid_ref[0]
    mu = jnp.mean(x, axis=-1, keepdims=True)
    var = jnp.mean((x - mu) * (x - mu), axis=-1, keepdims=True)
    ln = (x - mu) / jnp.sqrt(var + 1e-6)
    emb = emb_ref[0]
    shift = emb[:, :DIM]
    scale = emb[:, DIM:2 * DIM]
    out_ref[0] = (ln * (1.0 + scale) + shift).astype(BF)


def _ln_mod(hidden, emb):
    TB = 256
    return _pc(
        _ln_kernel,
        grid=(2, S // TB),
        in_specs=[
            pl.BlockSpec((1, TB, DIM), lambda b, t: (b, t, 0)),
            pl.BlockSpec((1, 1, 3 * DIM), lambda b, t: (b, 0, 0)),
        ],
        out_specs=pl.BlockSpec((1, TB, DIM), lambda b, t: (b, t, 0)),
        out_shape=jax.ShapeDtypeStruct((2, S, DIM), BF),
    )(hidden, emb.reshape(2, 1, 3 * DIM))


def _sel_kernel(hid_ref, temb_ref, w1_ref, wc_ref, b1_ref, w2_ref, b2_ref,
                mask_ref):
    x = hid_ref[0]
    cond = temb_ref[0] + jnp.mean(x[:TL], axis=0, keepdims=True)
    cond8 = jnp.concatenate([cond] * 8, axis=0)
    cdot = _bdot(cond8, wc_ref[...])[:1]
    h = _bdot(x[TL:], w1_ref[...]) + cdot + b1_ref[...]
    h = jax.nn.silu(h)
    lg = _bdot(h, w2_ref[...])[:, :1] + b2_ref[0]
    m = (jax.nn.sigmoid(lg / TAU) > 0.5).astype(F32)
    mask_ref[0] = jnp.broadcast_to(m, (IL, 128))


def _sel_mask(hidden, temb, sel_w1, sel_wc, sel_b1, sel_w2, sel_b2):
    w2p = jnp.pad(sel_w2, ((0, 0), (0, 127)))
    return _pc(
        _sel_kernel,
        grid=(2,),
        in_specs=[
            pl.BlockSpec((1, S, DIM), lambda b: (b, 0, 0)),
            pl.BlockSpec((1, 1, DIM), lambda b: (b, 0, 0)),
            pl.BlockSpec((DIM, 128), lambda b: (0, 0)),
            pl.BlockSpec((DIM, 128), lambda b: (0, 0)),
            pl.BlockSpec((1, 128), lambda b: (0, 0)),
            pl.BlockSpec((128, 128), lambda b: (0, 0)),
            pl.BlockSpec(memory_space=pltpu.SMEM),
        ],
        out_specs=pl.BlockSpec((1, IL, 128), lambda b: (b, 0, 0)),
        out_shape=jax.ShapeDtypeStruct((2, IL, 128), F32),
    )(hidden, temb.reshape(2, 1, DIM), sel_w1, sel_wc,
      sel_b1.reshape(1, 128), w2p, sel_b2)


def _kv_kernel(norm_ref, wk_ref, wv_ref, bk_ref, bv_ref, ks_ref,
               kout_ref, vout_ref):
    xbf = norm_ref[0]
    k = jnp.dot(xbf, wk_ref[...].astype(BF), preferred_element_type=F32)
    k = k + bk_ref[...]
    for hh in range(4):
        kh = k[:, hh * HD:(hh + 1) * HD]
        m = jnp.mean(kh * kh, axis=-1, keepdims=True)
        khn = kh / jnp.sqrt(m + 1e-6) * ks_ref[...]
        kout_ref[0, :, hh * HD:(hh + 1) * HD] = khn.astype(BF)
    v = jnp.dot(xbf, wv_ref[...].astype(BF), preferred_element_type=F32)
    vout_ref[0] = (v + bv_ref[...]).astype(BF)


def _kv(norm_bf, Wk, bk, Wv, bv, k_norm_scale):
    NB = 512
    return _pc(
        _kv_kernel,
        grid=(2, DIM // NB),
        in_specs=[
            pl.BlockSpec((1, S, DIM), lambda b, n: (b, 0, 0)),
            pl.BlockSpec((DIM, NB), lambda b, n: (0, n)),
            pl.BlockSpec((DIM, NB), lambda b, n: (0, n)),
            pl.BlockSpec((1, NB), lambda b, n: (0, n)),
            pl.BlockSpec((1, NB), lambda b, n: (0, n)),
            pl.BlockSpec((1, HD), lambda b, n: (0, 0)),
        ],
        out_specs=[
            pl.BlockSpec((1, S, NB), lambda b, n: (b, 0, n)),
            pl.BlockSpec((1, S, NB), lambda b, n: (b, 0, n)),
        ],
        out_shape=[
            jax.ShapeDtypeStruct((2, S, DIM), BF),
            jax.ShapeDtypeStruct((2, S, DIM), BF),
        ],
    )(norm_bf, Wk, Wv, bk.reshape(1, DIM), bv.reshape(1, DIM),
      k_norm_scale.reshape(1, HD))


def _attn_kernel(xq_ref, wq_ref, bq_ref, qs_ref, k_ref, v_ref, out_ref):
    xbf = xq_ref[0]
    q = jnp.dot(xbf, wq_ref[...].astype(BF), preferred_element_type=F32)
    q = q + bq_ref[...]
    scale = 1.0 / jnp.sqrt(128.0)
    for h in range(HEADS):
        qh = q[:, h * HD:(h + 1) * HD]
        m = jnp.mean(qh * qh, axis=-1, keepdims=True)
        qh = qh / jnp.sqrt(m + 1e-6) * qs_ref[...]
        kh = k_ref[0, :, h * HD:(h + 1) * HD]
        lg = jax.lax.dot_general(qh.astype(BF), kh,
                                 (((1,), (1,)), ((), ())),
                                 preferred_element_type=F32) * scale
        mx = jnp.max(lg, axis=-1, keepdims=True)
        e = jnp.exp(lg - mx)
        p = e / jnp.sum(e, axis=-1, keepdims=True)
        vh = v_ref[0, :, h * HD:(h + 1) * HD]
        ah = jnp.dot(p.astype(BF), vh, preferred_element_type=F32)
        out_ref[0, :, h * HD:(h + 1) * HD] = ah.astype(BF)


def _attention(xq_bf, Wq, bq, q_norm_scale, k_bf, v_bf):
    TB = 256
    return _pc(
        _attn_kernel,
        grid=(2, S // TB),
        in_specs=[
            pl.BlockSpec((1, TB, DIM), lambda b, t: (b, t, 0)),
            pl.BlockSpec((DIM, DIM), lambda b, t: (0, 0)),
            pl.BlockSpec((1, DIM), lambda b, t: (0, 0)),
            pl.BlockSpec((1, HD), lambda b, t: (0, 0)),
            pl.BlockSpec((1, S, DIM), lambda b, t: (b, 0, 0)),
            pl.BlockSpec((1, S, DIM), lambda b, t: (b, 0, 0)),
        ],
        out_specs=pl.BlockSpec((1, TB, DIM), lambda b, t: (b, t, 0)),
        out_shape=jax.ShapeDtypeStruct((2, S, DIM), BF),
    )(xq_bf, Wq, bq.reshape(1, DIM), q_norm_scale.reshape(1, HD), k_bf, v_bf)


_HB = 512
_NA = DIM // _HB
_RC = 256


def _mlp_kernel(x_ref, at_ref, mw_ref, mb_ref, w2_ref, b_ref, out_ref):
    h = pl.program_id(0)

    @pl.when(h == 0)
    def _init():
        out_ref[...] = jnp.broadcast_to(b_ref[...], out_ref.shape)

    w2bf = w2_ref[...].astype(BF)

    @pl.when(h < _NA)
    def _attn_part():
        for r in range((2 * S) // _RC):
            a = at_ref[r * _RC:(r + 1) * _RC]
            out_ref[r * _RC:(r + 1) * _RC] += jnp.dot(
                a, w2bf, preferred_element_type=F32)

    @pl.when(h >= _NA)
    def _mlp_part():
        mwbf = mw_ref[...].astype(BF)
        for r in range((2 * S) // _RC):
            x = x_ref[r * _RC:(r + 1) * _RC]
            g = jnp.dot(x, mwbf, preferred_element_type=F32) + mb_ref[...]
            g = jax.nn.gelu(g, approximate=True)
            out_ref[r * _RC:(r + 1) * _RC] += jnp.dot(
                g.astype(BF), w2bf, preferred_element_type=F32)


def _mlp(x_bf, at_bf, mlp_w, mlp_b, out_w, out_b):
    return _pc(
        _mlp_kernel,
        grid=(_NA + MLPH // _HB,),
        in_specs=[
            pl.BlockSpec((2 * S, DIM), lambda i: (0, 0)),
            pl.BlockSpec((2 * S, _HB),
                         lambda i: (0, jnp.where(i < _NA, i, 0))),
            pl.BlockSpec((DIM, _HB),
                         lambda i: (0, jnp.where(i < _NA, 0, i - _NA))),
            pl.BlockSpec((1, _HB),
                         lambda i: (0, jnp.where(i < _NA, 0, i - _NA))),
            pl.BlockSpec((_HB, DIM), lambda i: (i, 0)),
            pl.BlockSpec((1, DIM), lambda i: (0, 0)),
        ],
        out_specs=pl.BlockSpec((2 * S, DIM), lambda i: (0, 0)),
        out_shape=jax.ShapeDtypeStruct((2 * S, DIM), F32),
    )(x_bf, at_bf, mlp_w, mlp_b.reshape(1, MLPH), out_w,
      out_b.reshape(1, DIM))


def _fin_kernel(hid_ref, ym_ref, emb_ref, mask_ref, out_ref):
    y = ym_ref[0]
    m = mask_ref[0, :, :1]
    gate = emb_ref[0][:, 2 * DIM:]
    out_ref[0] = hid_ref[0] + gate * (y * m)


def _finish(hidden, ymlp, emb, maskb):
    TB = 256
    return _pc(
        _fin_kernel,
        grid=(2, S // TB),
        in_specs=[
            pl.BlockSpec((1, TB, DIM), lambda b, t: (b, t, 0)),
            pl.BlockSpec((1, TB, DIM), lambda b, t: (b, t, 0)),
            pl.BlockSpec((1, 1, 3 * DIM), lambda b, t: (b, 0, 0)),
            pl.BlockSpec((1, TB, 128), lambda b, t: (b, t, 0)),
        ],
        out_specs=pl.BlockSpec((1, TB, DIM), lambda b, t: (b, t, 0)),
        out_shape=jax.ShapeDtypeStruct((2, S, DIM), F32),
    )(hidden, ymlp, emb.reshape(2, 1, 3 * DIM), maskb)


def kernel(hidden_states, temb, adaln_w, adaln_b, Wq, bq, Wk, bk, Wv, bv,
           q_norm_scale, k_norm_scale, sel_w1, sel_wc, sel_b1, sel_w2, sel_b2,
           mlp_w, mlp_b, out_w, out_b):
    adaln_w = adaln_w.astype(BF)
    Wq = Wq.astype(BF)
    Wk = Wk.astype(BF)
    Wv = Wv.astype(BF)
    sel_w1 = sel_w1.astype(BF)
    sel_wc = sel_wc.astype(BF)
    mlp_w = mlp_w.astype(BF)
    out_w = out_w.astype(BF)
    emb = _adaln(temb, adaln_w, adaln_b)
    norm_bf = _ln_mod(hidden_states, emb)
    mask_img = _sel_mask(hidden_states, temb, sel_w1, sel_wc, sel_b1,
                         sel_w2, sel_b2)
    maskb = jnp.concatenate(
        [jnp.ones((2, TL, 128), F32), mask_img], axis=1)
    k_bf, v_bf = _kv(norm_bf, Wk, bk, Wv, bv, k_norm_scale)
    attn_bf = _attention(norm_bf, Wq, bq, q_norm_scale, k_bf, v_bf)
    ymlp = _mlp(norm_bf.reshape(2 * S, DIM), attn_bf.reshape(2 * S, DIM),
                mlp_w, mlp_b, out_w, out_b)
    return _finish(hidden_states, ymlp.reshape(2, S, DIM), emb, maskb)


# speedup vs baseline: 1.2557x; 1.2557x over previous
import jax
import jax.numpy as jnp
from jax.experimental import pallas as pl
from jax.experimental.pallas import tpu as pltpu

DIM = 2048
HEADS = 16
HD = 128
MLPH = 8192
TL = 256
IL = 1024
S = TL + IL
TAU = 5.0
BF = jnp.bfloat16
F32 = jnp.float32

_INTERPRET = False


def _pc(*args, **kwargs):
    return pl.pallas_call(*args, interpret=_INTERPRET, **kwargs)


def _bdot(a, b):
    return jnp.dot(a.astype(BF), b.astype(BF), preferred_element_type=F32)


def _adaln_kernel(temb_ref, w_ref, b_ref, out_ref):
    t = jax.nn.silu(temb_ref[...])
    t8 = jnp.concatenate([t, t, t, t], axis=0)
    r = _bdot(t8, w_ref[...])
    out_ref[...] = r[:2, :] + b_ref[...]


def _adaln(temb, adaln_w, adaln_b):
    NB = 512
    n = (3 * DIM) // NB
    return _pc(
        _adaln_kernel,
        grid=(n,),
        in_specs=[
            pl.BlockSpec((2, DIM), lambda i: (0, 0)),
            pl.BlockSpec((DIM, NB), lambda i: (0, i)),
            pl.BlockSpec((1, NB), lambda i: (0, i)),
        ],
        out_specs=pl.BlockSpec((2, NB), lambda i: (0, i)),
        out_shape=jax.ShapeDtypeStruct((2, 3 * DIM), F32),
    )(temb, adaln_w, adaln_b.reshape(1, 3 * DIM))


def _ln_kernel(hid_ref, emb_ref, out_ref):
    x = hid_ref[0]
    mu = jnp.mean(x, axis=-1, keepdims=True)
    var = jnp.mean((x - mu) * (x - mu), axis=-1, keepdims=True)
    ln = (x - mu) / jnp.sqrt(var + 1e-6)
    emb = emb_ref[0]
    shift = emb[:, :DIM]
    scale = emb[:, DIM:2 * DIM]
    out_ref[0] = (ln * (1.0 + scale) + shift).astype(BF)


def _ln_mod(hidden, emb):
    TB = 256
    return _pc(
        _ln_kernel,
        grid=(2, S // TB),
        in_specs=[
            pl.BlockSpec((1, TB, DIM), lambda b, t: (b, t, 0)),
            pl.BlockSpec((1, 1, 3 * DIM), lambda b, t: (b, 0, 0)),
        ],
        out_specs=pl.BlockSpec((1, TB, DIM), lambda b, t: (b, t, 0)),
        out_shape=jax.ShapeDtypeStruct((2, S, DIM), BF),
    )(hidden, emb.reshape(2, 1, 3 * DIM))


def _sel_kernel(hid_ref, temb_ref, w1_ref, wc_ref, b1_ref, w2_ref, b2_ref,
                mask_ref):
    x = hid_ref[0]
    cond = temb_ref[0] + jnp.mean(x[:TL], axis=0, keepdims=True)
    cond8 = jnp.concatenate([cond] * 8, axis=0)
    cdot = _bdot(cond8, wc_ref[...])[:1]
    h = _bdot(x[TL:], w1_ref[...]) + cdot + b1_ref[...]
    h = jax.nn.silu(h)
    lg = _bdot(h, w2_ref[...])[:, :1] + b2_ref[0]
    m = (jax.nn.sigmoid(lg / TAU) > 0.5).astype(F32)
    mask_ref[0] = jnp.broadcast_to(m, (IL, 128))


def _sel_mask(hidden, temb, sel_w1, sel_wc, sel_b1, sel_w2, sel_b2):
    w2p = jnp.pad(sel_w2, ((0, 0), (0, 127)))
    return _pc(
        _sel_kernel,
        grid=(2,),
        in_specs=[
            pl.BlockSpec((1, S, DIM), lambda b: (b, 0, 0)),
            pl.BlockSpec((1, 1, DIM), lambda b: (b, 0, 0)),
            pl.BlockSpec((DIM, 128), lambda b: (0, 0)),
            pl.BlockSpec((DIM, 128), lambda b: (0, 0)),
            pl.BlockSpec((1, 128), lambda b: (0, 0)),
            pl.BlockSpec((128, 128), lambda b: (0, 0)),
            pl.BlockSpec(memory_space=pltpu.SMEM),
        ],
        out_specs=pl.BlockSpec((1, IL, 128), lambda b: (b, 0, 0)),
        out_shape=jax.ShapeDtypeStruct((2, IL, 128), F32),
    )(hidden, temb.reshape(2, 1, DIM), sel_w1, sel_wc,
      sel_b1.reshape(1, 128), w2p, sel_b2)


def _kv_kernel(norm_ref, wk_ref, wv_ref, bk_ref, bv_ref, ks_ref,
               kout_ref, vout_ref):
    xbf = norm_ref[0]
    k = jnp.dot(xbf, wk_ref[...].astype(BF), preferred_element_type=F32)
    k = k + bk_ref[...]
    for hh in range(4):
        kh = k[:, hh * HD:(hh + 1) * HD]
        m = jnp.mean(kh * kh, axis=-1, keepdims=True)
        khn = kh / jnp.sqrt(m + 1e-6) * ks_ref[...]
        kout_ref[0, :, hh * HD:(hh + 1) * HD] = khn.astype(BF)
    v = jnp.dot(xbf, wv_ref[...].astype(BF), preferred_element_type=F32)
    vout_ref[0] = (v + bv_ref[...]).astype(BF)


def _kv(norm_bf, Wk, bk, Wv, bv, k_norm_scale):
    NB = 512
    return _pc(
        _kv_kernel,
        grid=(2, DIM // NB),
        in_specs=[
            pl.BlockSpec((1, S, DIM), lambda b, n: (b, 0, 0)),
            pl.BlockSpec((DIM, NB), lambda b, n: (0, n)),
            pl.BlockSpec((DIM, NB), lambda b, n: (0, n)),
            pl.BlockSpec((1, NB), lambda b, n: (0, n)),
            pl.BlockSpec((1, NB), lambda b, n: (0, n)),
            pl.BlockSpec((1, HD), lambda b, n: (0, 0)),
        ],
        out_specs=[
            pl.BlockSpec((1, S, NB), lambda b, n: (b, 0, n)),
            pl.BlockSpec((1, S, NB), lambda b, n: (b, 0, n)),
        ],
        out_shape=[
            jax.ShapeDtypeStruct((2, S, DIM), BF),
            jax.ShapeDtypeStruct((2, S, DIM), BF),
        ],
    )(norm_bf, Wk, Wv, bk.reshape(1, DIM), bv.reshape(1, DIM),
      k_norm_scale.reshape(1, HD))


def _attn_kernel(xq_ref, wq_ref, bq_ref, qs_ref, k_ref, v_ref, out_ref):
    xbf = xq_ref[0]
    q = jnp.dot(xbf, wq_ref[...].astype(BF), preferred_element_type=F32)
    q = q + bq_ref[...]
    scale = 1.0 / jnp.sqrt(128.0)
    for h in range(HEADS):
        qh = q[:, h * HD:(h + 1) * HD]
        m = jnp.mean(qh * qh, axis=-1, keepdims=True)
        qh = qh / jnp.sqrt(m + 1e-6) * qs_ref[...]
        kh = k_ref[0, :, h * HD:(h + 1) * HD]
        lg = jax.lax.dot_general(qh.astype(BF), kh,
                                 (((1,), (1,)), ((), ())),
                                 preferred_element_type=F32) * scale
        mx = jnp.max(lg, axis=-1, keepdims=True)
        e = jnp.exp(lg - mx)
        p = e / jnp.sum(e, axis=-1, keepdims=True)
        vh = v_ref[0, :, h * HD:(h + 1) * HD]
        ah = jnp.dot(p.astype(BF), vh, preferred_element_type=F32)
        out_ref[0, :, h * HD:(h + 1) * HD] = ah.astype(BF)


def _attention(xq_bf, Wq, bq, q_norm_scale, k_bf, v_bf):
    TB = 256
    return _pc(
        _attn_kernel,
        grid=(2, S // TB),
        in_specs=[
            pl.BlockSpec((1, TB, DIM), lambda b, t: (b, t, 0)),
            pl.BlockSpec((DIM, DIM), lambda b, t: (0, 0)),
            pl.BlockSpec((1, DIM), lambda b, t: (0, 0)),
            pl.BlockSpec((1, HD), lambda b, t: (0, 0)),
            pl.BlockSpec((1, S, DIM), lambda b, t: (b, 0, 0)),
            pl.BlockSpec((1, S, DIM), lambda b, t: (b, 0, 0)),
        ],
        out_specs=pl.BlockSpec((1, TB, DIM), lambda b, t: (b, t, 0)),
        out_shape=jax.ShapeDtypeStruct((2, S, DIM), BF),
    )(xq_bf, Wq, bq.reshape(1, DIM), q_norm_scale.reshape(1, HD), k_bf, v_bf)


_HB = 512
_NA = DIM // _HB
_RC = 512


def _mlp_kernel(x_ref, at_ref, mw_ref, mb_ref, w2_ref, b_ref, out_ref):
    h = pl.program_id(0)

    @pl.when(h == 0)
    def _init():
        out_ref[...] = jnp.broadcast_to(b_ref[...], out_ref.shape)

    w2bf = w2_ref[...].astype(BF)

    @pl.when(h < _NA)
    def _attn_part():
        for r in range((2 * S) // _RC):
            a = at_ref[r * _RC:(r + 1) * _RC]
            out_ref[r * _RC:(r + 1) * _RC] += jnp.dot(
                a, w2bf, preferred_element_type=F32)

    @pl.when(h >= _NA)
    def _mlp_part():
        mwbf = mw_ref[...].astype(BF)
        for r in range((2 * S) // _RC):
            x = x_ref[r * _RC:(r + 1) * _RC]
            g = jnp.dot(x, mwbf, preferred_element_type=F32) + mb_ref[...]
            g = jax.nn.gelu(g, approximate=True)
            out_ref[r * _RC:(r + 1) * _RC] += jnp.dot(
                g.astype(BF), w2bf, preferred_element_type=F32)


def _mlp(x_bf, at_bf, mlp_w, mlp_b, out_w, out_b):
    return _pc(
        _mlp_kernel,
        grid=(_NA + MLPH // _HB,),
        in_specs=[
            pl.BlockSpec((2 * S, DIM), lambda i: (0, 0)),
            pl.BlockSpec((2 * S, _HB),
                         lambda i: (0, jnp.where(i < _NA, i, 0))),
            pl.BlockSpec((DIM, _HB),
                         lambda i: (0, jnp.where(i < _NA, 0, i - _NA))),
            pl.BlockSpec((1, _HB),
                         lambda i: (0, jnp.where(i < _NA, 0, i - _NA))),
            pl.BlockSpec((_HB, DIM), lambda i: (i, 0)),
            pl.BlockSpec((1, DIM), lambda i: (0, 0)),
        ],
        out_specs=pl.BlockSpec((2 * S, DIM), lambda i: (0, 0)),
        out_shape=jax.ShapeDtypeStruct((2 * S, DIM), F32),
    )(x_bf, at_bf, mlp_w, mlp_b.reshape(1, MLPH), out_w,
      out_b.reshape(1, DIM))


def _fin_kernel(hid_ref, ym_ref, emb_ref, mask_ref, out_ref):
    y = ym_ref[0]
    m = mask_ref[0, :, :1]
    gate = emb_ref[0][:, 2 * DIM:]
    out_ref[0] = hid_ref[0] + gate * (y * m)


def _finish(hidden, ymlp, emb, maskb):
    TB = 256
    return _pc(
        _fin_kernel,
        grid=(2, S // TB),
        in_specs=[
            pl.BlockSpec((1, TB, DIM), lambda b, t: (b, t, 0)),
            pl.BlockSpec((1, TB, DIM), lambda b, t: (b, t, 0)),
            pl.BlockSpec((1, 1, 3 * DIM), lambda b, t: (b, 0, 0)),
            pl.BlockSpec((1, TB, 128), lambda b, t: (b, t, 0)),
        ],
        out_specs=pl.BlockSpec((1, TB, DIM), lambda b, t: (b, t, 0)),
        out_shape=jax.ShapeDtypeStruct((2, S, DIM), F32),
    )(hidden, ymlp, emb.reshape(2, 1, 3 * DIM), maskb)


def kernel(hidden_states, temb, adaln_w, adaln_b, Wq, bq, Wk, bk, Wv, bv,
           q_norm_scale, k_norm_scale, sel_w1, sel_wc, sel_b1, sel_w2, sel_b2,
           mlp_w, mlp_b, out_w, out_b):
    emb = _adaln(temb, adaln_w, adaln_b)
    norm_bf = _ln_mod(hidden_states, emb)
    mask_img = _sel_mask(hidden_states, temb, sel_w1, sel_wc, sel_b1,
                         sel_w2, sel_b2)
    maskb = jnp.concatenate(
        [jnp.ones((2, TL, 128), F32), mask_img], axis=1)
    k_bf, v_bf = _kv(norm_bf, Wk, bk, Wv, bv, k_norm_scale)
    attn_bf = _attention(norm_bf, Wq, bq, q_norm_scale, k_bf, v_bf)
    ymlp = _mlp(norm_bf.reshape(2 * S, DIM), attn_bf.reshape(2 * S, DIM),
                mlp_w, mlp_b, out_w, out_b)
    return _finish(hidden_states, ymlp.reshape(2, S, DIM), emb, maskb)
